# baseline (speedup 1.0000x reference)
import functools

import jax
import jax.numpy as jnp
from jax import lax
from jax.experimental import pallas as pl
from jax.experimental.pallas import tpu as pltpu
from jax.experimental.pallas import tpu_sc as plsc

N_NODES = 100000
N_EDGES = 6400000
HID = 8
Y_DIM = 16

NC = 2
NS = 16
NW = NC * NS
RW = 128

ROWS = N_EDGES // RW
FULL_ROWS = 1536

KA = 64
TA = FULL_ROWS // KA
KB = 24
TB = FULL_ROWS // KB
MSTG = 2096

NACC = 100608
SLICE = NACC // NS

_MESH = plsc.VectorSubcoreMesh(core_axis_name="c", subcore_axis_name="s")


def _zero_fill(buf, n):
    def body(i, _):
        buf[pl.ds(i * 16, 16)] = jnp.zeros((16,), jnp.float32)
        return 0
    lax.fori_loop(0, n // 16, body, 0)


def _worker_rows(c, s):
    w = c * NS + s
    return 8 * (195 * w + jnp.minimum(w, 10))


@functools.partial(
    pl.kernel,
    out_type=jax.ShapeDtypeStruct((NC * NACC,), jnp.float32),
    mesh=_MESH,
    scratch_types=[
        pltpu.VMEM((KA, RW), jnp.int32),
        pltpu.VMEM((KA, RW), jnp.int32),
        pltpu.VMEM((RW,), jnp.float32),
        pltpu.VMEM((SLICE,), jnp.float32),
        pltpu.VMEM_SHARED((NACC,), jnp.float32),
        pltpu.SemaphoreType.DMA,
        pltpu.SemaphoreType.DMA,
    ],
)
def _deg_kernel(ei_hbm, degp_hbm, idx_a, idx_b, ones_v, stage_v, acc_sh,
                ssem, isem):
    c = lax.axis_index("c")
    s = lax.axis_index("s")
    w = c * NS + s

    for i in range(RW // 16):
        ones_v[pl.ds(i * 16, 16)] = jnp.full((16,), 1.0, jnp.float32)
    _zero_fill(stage_v, SLICE)
    pltpu.sync_copy(stage_v, acc_sh.at[pl.ds(s * SLICE, SLICE)])
    plsc.subcore_barrier()

    row0 = _worker_rows(c, s)

    def _fire(buf, nrows):
        for j in range(nrows):
            pltpu.async_copy(ones_v, acc_sh.at[buf.at[j]], ssem, add=True)

    def _drain(buf, nrows):
        for j in range(nrows):
            pltpu.make_async_copy(ones_v, acc_sh.at[pl.ds(0, RW)], ssem).wait()

    def _dma_start(buf, row):
        pltpu.async_copy(ei_hbm.at[1, pl.ds(row, KA)], buf, isem)

    def _dma_wait(buf):
        pltpu.make_async_copy(ei_hbm.at[1, pl.ds(row0, KA)], buf, isem).wait()

    _dma_start(idx_a, row0)

    def pair(g, _):
        _dma_wait(idx_a)

        @pl.when(g > 0)
        def _():
            _drain(idx_b, KA)
        _dma_start(idx_b, row0 + (2 * g + 1) * KA)
        _fire(idx_a, KA)
        _dma_wait(idx_b)
        _drain(idx_a, KA)

        @pl.when(g < TA // 2 - 1)
        def _():
            _dma_start(idx_a, row0 + (2 * g + 2) * KA)
        _fire(idx_b, KA)
        return 0

    lax.fori_loop(0, TA // 2, pair, 0)
    _drain(idx_b, KA)

    pltpu.sync_copy(ei_hbm.at[1, pl.ds(row0 + FULL_ROWS, 24)],
                    idx_a.at[pl.ds(0, 24)])
    _fire(idx_a, 24)
    _drain(idx_a, 24)

    @pl.when(w < 10)
    def _():
        pltpu.sync_copy(ei_hbm.at[1, pl.ds(row0 + FULL_ROWS + 24, 8)],
                        idx_a.at[pl.ds(0, 8)])
        _fire(idx_a, 8)
        _drain(idx_a, 8)

    plsc.subcore_barrier()
    pltpu.sync_copy(acc_sh.at[pl.ds(s * SLICE, SLICE)], stage_v)
    pltpu.sync_copy(stage_v, degp_hbm.at[pl.ds(c * NACC + s * SLICE, SLICE)])


@functools.partial(
    pl.kernel,
    out_type=jax.ShapeDtypeStruct((NC * NACC,), jnp.float32),
    mesh=_MESH,
    compiler_params=pltpu.CompilerParams(needs_layout_passes=False),
    scratch_types=[
        pltpu.VMEM((KB, RW), jnp.int32),
        pltpu.VMEM((KB, RW), jnp.int32),
        pltpu.VMEM((KB, RW), jnp.float32),
        pltpu.VMEM((KB, RW), jnp.int32),
        pltpu.VMEM((KB, RW), jnp.int32),
        pltpu.VMEM((KB, RW), jnp.float32),
        pltpu.VMEM((N_NODES,), jnp.float32),
        pltpu.VMEM((MSTG,), jnp.float32),
        pltpu.VMEM_SHARED((NACC,), jnp.float32),
        pltpu.SemaphoreType.DMA,
        pltpu.SemaphoreType.DMA,
    ],
)
def _msg_kernel(ei_hbm, v_hbm, accp_hbm,
                sidx_a, didx_a, vals_a, sidx_b, didx_b, vals_b,
                v_vmem, stage_v, acc_sh, ssem, isem):
    c = lax.axis_index("c")
    s = lax.axis_index("s")
    w = c * NS + s

    _zero_fill(stage_v, MSTG)
    for q in range(SLICE // MSTG):
        pltpu.sync_copy(stage_v, acc_sh.at[pl.ds(s * SLICE + q * MSTG, MSTG)])
    pltpu.sync_copy(v_hbm, v_vmem)
    plsc.subcore_barrier()

    row0 = _worker_rows(c, s)

    def _gather(sidx, vals, nrows):
        for r in range(nrows):
            for u in range(RW // 16):
                idx = sidx[r, pl.ds(u * 16, 16)]
                vals[r, pl.ds(u * 16, 16)] = plsc.load_gather(v_vmem, [idx])

    def _fire(vals, didx, nrows):
        for j in range(nrows):
            pltpu.async_copy(vals.at[j], acc_sh.at[didx.at[j]], ssem, add=True)

    def _drain(nrows):
        for j in range(nrows):
            pltpu.make_async_copy(vals_a.at[0], acc_sh.at[pl.ds(0, RW)],
                                  ssem).wait()

    def _dma_start(sidx, didx, row):
        pltpu.async_copy(ei_hbm.at[0, pl.ds(row, KB)], sidx, isem)
        pltpu.async_copy(ei_hbm.at[1, pl.ds(row, KB)], didx, isem)

    def _dma_wait(sidx, didx):
        pltpu.make_async_copy(ei_hbm.at[0, pl.ds(row0, KB)], sidx, isem).wait()
        pltpu.make_async_copy(ei_hbm.at[1, pl.ds(row0, KB)], didx, isem).wait()

    _dma_start(sidx_a, didx_a, row0)

    def pair(g, _):
        _dma_wait(sidx_a, didx_a)

        @pl.when(g > 0)
        def _():
            _drain(KB)
        _gather(sidx_a, vals_a, KB)
        _dma_start(sidx_b, didx_b, row0 + (2 * g + 1) * KB)
        _fire(vals_a, didx_a, KB)
        _dma_wait(sidx_b, didx_b)
        _gather(sidx_b, vals_b, KB)
        _drain(KB)

        @pl.when(g < TB // 2 - 1)
        def _():
            _dma_start(sidx_a, didx_a, row0 + (2 * g + 2) * KB)
        _fire(vals_b, didx_b, KB)
        return 0

    lax.fori_loop(0, TB // 2, pair, 0)
    _drain(KB)

    pltpu.sync_copy(ei_hbm.at[0, pl.ds(row0 + FULL_ROWS, 24)],
                    sidx_a.at[pl.ds(0, 24)])
    pltpu.sync_copy(ei_hbm.at[1, pl.ds(row0 + FULL_ROWS, 24)],
                    didx_a.at[pl.ds(0, 24)])
    _gather(sidx_a, vals_a, 24)
    _fire(vals_a, didx_a, 24)
    _drain(24)

    @pl.when(w < 10)
    def _():
        pltpu.sync_copy(ei_hbm.at[0, pl.ds(row0 + FULL_ROWS + 24, 8)],
                        sidx_a.at[pl.ds(0, 8)])
        pltpu.sync_copy(ei_hbm.at[1, pl.ds(row0 + FULL_ROWS + 24, 8)],
                        didx_a.at[pl.ds(0, 8)])
        _gather(sidx_a, vals_a, 8)
        _fire(vals_a, didx_a, 8)
        _drain(8)

    plsc.subcore_barrier()
    for q in range(SLICE // MSTG):
        off = s * SLICE + q * MSTG
        pltpu.sync_copy(acc_sh.at[pl.ds(off, MSTG)], stage_v)
        pltpu.sync_copy(stage_v, accp_hbm.at[pl.ds(c * NACC + off, MSTG)])


BC = 16000
NBLK = N_NODES * HID // BC


def _head_body(w_ref, f_ref, b_ref, o_ref):
    @pl.when(pl.program_id(0) == 0)
    def _():
        o_ref[...] = b_ref[...]
    o_ref[...] += jnp.sum(w_ref[...] * f_ref[...], axis=1, keepdims=True)


_head_call = pl.pallas_call(
    _head_body,
    grid=(NBLK,),
    in_specs=[
        pl.BlockSpec((Y_DIM, BC), lambda i: (0, i)),
        pl.BlockSpec((1, BC), lambda i: (0, i)),
        pl.BlockSpec((Y_DIM, 1), lambda i: (0, 0)),
    ],
    out_specs=pl.BlockSpec((Y_DIM, 1), lambda i: (0, 0)),
    out_shape=jax.ShapeDtypeStruct((Y_DIM, 1), jnp.float32),
)


def kernel(x, edge_index, W1, b1, Wreg, breg):
    ei = edge_index.astype(jnp.int32).reshape(2, ROWS, RW)

    degp = _deg_kernel(ei).reshape(NC, NACC)
    deg = degp[0, :N_NODES] + degp[1, :N_NODES] + 1.0
    dis = lax.rsqrt(deg)
    xf = x[:, 0]
    v = xf * dis

    accp = _msg_kernel(ei, v).reshape(NC, NACC)
    s = dis * (accp[0, :N_NODES] + accp[1, :N_NODES]) + xf / deg

    ar = jnp.arange(N_NODES * HID, dtype=jnp.int32)[None, :]
    flat = jax.nn.relu(s[ar // HID] * W1[0, ar % HID] + b1[ar % HID])
    y = _head_call(Wreg, flat, breg.reshape(Y_DIM, 1))
    return y[:, 0]


# speedup vs baseline: 32.1564x; 32.1564x over previous
import functools

import jax
import jax.numpy as jnp
from jax import lax
from jax.experimental import pallas as pl
from jax.experimental.pallas import tpu as pltpu
from jax.experimental.pallas import tpu_sc as plsc

N_NODES = 100000
N_EDGES = 6400000
HID = 8
Y_DIM = 16

NC = 2
NS = 16
NW = NC * NS
RW = 128

ROWS = N_EDGES // RW
FULL_ROWS = 1536

KA = 64
TA = FULL_ROWS // KA
KB = 24
TB = FULL_ROWS // KB
MSTG = 2096

NACC = 100608
SLICE = NACC // NS

_MESH = plsc.VectorSubcoreMesh(core_axis_name="c", subcore_axis_name="s")


def _zero_fill(buf, n):
    def body(i, _):
        buf[pl.ds(i * 16, 16)] = jnp.zeros((16,), jnp.float32)
        return 0
    lax.fori_loop(0, n // 16, body, 0)


def _worker_rows(c, s):
    w = c * NS + s
    return 8 * (195 * w + jnp.minimum(w, 10))


@functools.partial(
    pl.kernel,
    out_type=jax.ShapeDtypeStruct((NC * NACC,), jnp.float32),
    mesh=_MESH,
    scratch_types=[
        pltpu.VMEM((KA, RW), jnp.int32),
        pltpu.VMEM((KA, RW), jnp.int32),
        pltpu.VMEM((RW,), jnp.float32),
        pltpu.VMEM((SLICE,), jnp.float32),
        pltpu.VMEM_SHARED((NACC,), jnp.float32),
        pltpu.SemaphoreType.DMA,
        pltpu.SemaphoreType.DMA,
    ],
)
def _deg_kernel(ei_hbm, degp_hbm, idx_a, idx_b, ones_v, stage_v, acc_sh,
                ssem, isem):
    c = lax.axis_index("c")
    s = lax.axis_index("s")
    w = c * NS + s

    for i in range(RW // 16):
        ones_v[pl.ds(i * 16, 16)] = jnp.full((16,), 1.0, jnp.float32)
    _zero_fill(stage_v, SLICE)
    pltpu.sync_copy(stage_v, acc_sh.at[pl.ds(s * SLICE, SLICE)])
    plsc.subcore_barrier()

    row0 = _worker_rows(c, s)

    def _fire(buf, nrows):
        for j in range(nrows):
            pltpu.async_copy(ones_v, acc_sh.at[buf.at[j]], ssem, add=True)

    def _drain(buf, nrows):
        for j in range(nrows):
            pltpu.make_async_copy(ones_v, acc_sh.at[pl.ds(0, RW)], ssem).wait()

    def _dma_start(buf, row):
        pltpu.async_copy(ei_hbm.at[1, pl.ds(row, KA)], buf, isem)

    def _dma_wait(buf):
        pltpu.make_async_copy(ei_hbm.at[1, pl.ds(row0, KA)], buf, isem).wait()

    _dma_start(idx_a, row0)

    def pair(g, _):
        _dma_wait(idx_a)

        @pl.when(g > 0)
        def _():
            _drain(idx_b, KA)
        _dma_start(idx_b, row0 + (2 * g + 1) * KA)
        _fire(idx_a, KA)
        _dma_wait(idx_b)
        _drain(idx_a, KA)

        @pl.when(g < TA // 2 - 1)
        def _():
            _dma_start(idx_a, row0 + (2 * g + 2) * KA)
        _fire(idx_b, KA)
        return 0

    lax.fori_loop(0, TA // 2, pair, 0)
    _drain(idx_b, KA)

    pltpu.sync_copy(ei_hbm.at[1, pl.ds(row0 + FULL_ROWS, 24)],
                    idx_a.at[pl.ds(0, 24)])
    _fire(idx_a, 24)
    _drain(idx_a, 24)

    @pl.when(w < 10)
    def _():
        pltpu.sync_copy(ei_hbm.at[1, pl.ds(row0 + FULL_ROWS + 24, 8)],
                        idx_a.at[pl.ds(0, 8)])
        _fire(idx_a, 8)
        _drain(idx_a, 8)

    plsc.subcore_barrier()
    pltpu.sync_copy(acc_sh.at[pl.ds(s * SLICE, SLICE)], stage_v)
    pltpu.sync_copy(stage_v, degp_hbm.at[pl.ds(c * NACC + s * SLICE, SLICE)])


@functools.partial(
    pl.kernel,
    out_type=jax.ShapeDtypeStruct((NC * NACC,), jnp.float32),
    mesh=_MESH,
    compiler_params=pltpu.CompilerParams(needs_layout_passes=False),
    scratch_types=[
        pltpu.VMEM((KB, RW), jnp.int32),
        pltpu.VMEM((KB, RW), jnp.int32),
        pltpu.VMEM((KB, RW), jnp.float32),
        pltpu.VMEM((KB, RW), jnp.int32),
        pltpu.VMEM((KB, RW), jnp.int32),
        pltpu.VMEM((KB, RW), jnp.float32),
        pltpu.VMEM((N_NODES,), jnp.float32),
        pltpu.VMEM((MSTG,), jnp.float32),
        pltpu.VMEM_SHARED((NACC,), jnp.float32),
        pltpu.SemaphoreType.DMA,
        pltpu.SemaphoreType.DMA,
    ],
)
def _msg_kernel(ei_hbm, v_hbm, accp_hbm,
                sidx_a, didx_a, vals_a, sidx_b, didx_b, vals_b,
                v_vmem, stage_v, acc_sh, ssem, isem):
    c = lax.axis_index("c")
    s = lax.axis_index("s")
    w = c * NS + s

    _zero_fill(stage_v, MSTG)
    for q in range(SLICE // MSTG):
        pltpu.sync_copy(stage_v, acc_sh.at[pl.ds(s * SLICE + q * MSTG, MSTG)])
    pltpu.sync_copy(v_hbm, v_vmem)
    plsc.subcore_barrier()

    row0 = _worker_rows(c, s)

    def _gather(sidx, vals, nrows):
        for r in range(nrows):
            for u in range(RW // 16):
                idx = sidx[r, pl.ds(u * 16, 16)]
                vals[r, pl.ds(u * 16, 16)] = plsc.load_gather(v_vmem, [idx])

    def _fire(vals, didx, nrows):
        for j in range(nrows):
            pltpu.async_copy(vals.at[j], acc_sh.at[didx.at[j]], ssem, add=True)

    def _drain(nrows):
        for j in range(nrows):
            pltpu.make_async_copy(vals_a.at[0], acc_sh.at[pl.ds(0, RW)],
                                  ssem).wait()

    def _dma_start(sidx, didx, row):
        pltpu.async_copy(ei_hbm.at[0, pl.ds(row, KB)], sidx, isem)
        pltpu.async_copy(ei_hbm.at[1, pl.ds(row, KB)], didx, isem)

    def _dma_wait(sidx, didx):
        pltpu.make_async_copy(ei_hbm.at[0, pl.ds(row0, KB)], sidx, isem).wait()
        pltpu.make_async_copy(ei_hbm.at[1, pl.ds(row0, KB)], didx, isem).wait()

    _dma_start(sidx_a, didx_a, row0)

    def pair(g, _):
        _dma_wait(sidx_a, didx_a)

        @pl.when(g > 0)
        def _():
            _drain(KB)
        _gather(sidx_a, vals_a, KB)
        _dma_start(sidx_b, didx_b, row0 + (2 * g + 1) * KB)
        _fire(vals_a, didx_a, KB)
        _dma_wait(sidx_b, didx_b)
        _gather(sidx_b, vals_b, KB)
        _drain(KB)

        @pl.when(g < TB // 2 - 1)
        def _():
            _dma_start(sidx_a, didx_a, row0 + (2 * g + 2) * KB)
        _fire(vals_b, didx_b, KB)
        return 0

    lax.fori_loop(0, TB // 2, pair, 0)
    _drain(KB)

    pltpu.sync_copy(ei_hbm.at[0, pl.ds(row0 + FULL_ROWS, 24)],
                    sidx_a.at[pl.ds(0, 24)])
    pltpu.sync_copy(ei_hbm.at[1, pl.ds(row0 + FULL_ROWS, 24)],
                    didx_a.at[pl.ds(0, 24)])
    _gather(sidx_a, vals_a, 24)
    _fire(vals_a, didx_a, 24)
    _drain(24)

    @pl.when(w < 10)
    def _():
        pltpu.sync_copy(ei_hbm.at[0, pl.ds(row0 + FULL_ROWS + 24, 8)],
                        sidx_a.at[pl.ds(0, 8)])
        pltpu.sync_copy(ei_hbm.at[1, pl.ds(row0 + FULL_ROWS + 24, 8)],
                        didx_a.at[pl.ds(0, 8)])
        _gather(sidx_a, vals_a, 8)
        _fire(vals_a, didx_a, 8)
        _drain(8)

    plsc.subcore_barrier()
    for q in range(SLICE // MSTG):
        off = s * SLICE + q * MSTG
        pltpu.sync_copy(acc_sh.at[pl.ds(off, MSTG)], stage_v)
        pltpu.sync_copy(stage_v, accp_hbm.at[pl.ds(c * NACC + off, MSTG)])


FPW = 25600
SPW = FPW // HID
FPAD = NW * FPW


@functools.partial(
    pl.kernel,
    out_type=jax.ShapeDtypeStruct((1, FPAD), jnp.float32),
    mesh=_MESH,
    compiler_params=pltpu.CompilerParams(needs_layout_passes=False),
    scratch_types=[
        pltpu.VMEM((SPW,), jnp.float32),
        pltpu.VMEM((16,), jnp.float32),
        pltpu.VMEM((16,), jnp.float32),
        pltpu.VMEM((FPW,), jnp.float32),
    ],
)
def _flat_kernel(s_hbm, w1_hbm, b1_hbm, flat_hbm, s_v, w1_v, b1_v, f_v):
    c = lax.axis_index("c")
    s = lax.axis_index("s")
    w = c * NS + s

    pltpu.sync_copy(s_hbm.at[pl.ds(w * SPW, SPW)], s_v)
    pltpu.sync_copy(w1_hbm, w1_v)
    pltpu.sync_copy(b1_hbm, b1_v)
    w1r = w1_v[pl.ds(0, 16)]
    b1r = b1_v[pl.ds(0, 16)]
    rep = lax.iota(jnp.int32, 16) // HID

    def body(k, _):
        g = plsc.load_gather(s_v, [rep + 2 * k])
        f_v[pl.ds(16 * k, 16)] = jnp.maximum(g * w1r + b1r, 0.0)
        return 0

    lax.fori_loop(0, FPW // 16, body, 0)
    pltpu.sync_copy(f_v, flat_hbm.at[0, pl.ds(w * FPW, FPW)])


BC = 16000
NBLK = N_NODES * HID // BC


def _head_body(w_ref, f_ref, b_ref, o_ref):
    @pl.when(pl.program_id(0) == 0)
    def _():
        o_ref[...] = b_ref[...]
    o_ref[...] += jnp.sum(w_ref[...] * f_ref[...], axis=1, keepdims=True)


_head_call = pl.pallas_call(
    _head_body,
    grid=(NBLK,),
    in_specs=[
        pl.BlockSpec((Y_DIM, BC), lambda i: (0, i)),
        pl.BlockSpec((1, BC), lambda i: (0, i)),
        pl.BlockSpec((Y_DIM, 1), lambda i: (0, 0)),
    ],
    out_specs=pl.BlockSpec((Y_DIM, 1), lambda i: (0, 0)),
    out_shape=jax.ShapeDtypeStruct((Y_DIM, 1), jnp.float32),
)


def kernel(x, edge_index, W1, b1, Wreg, breg):
    ei = edge_index.astype(jnp.int32).reshape(2, ROWS, RW)

    degp = _deg_kernel(ei).reshape(NC, NACC)
    deg = degp[0, :N_NODES] + degp[1, :N_NODES] + 1.0
    dis = lax.rsqrt(deg)
    xf = x[:, 0]
    v = xf * dis

    accp = _msg_kernel(ei, v).reshape(NC, NACC)
    s = dis * (accp[0, :N_NODES] + accp[1, :N_NODES]) + xf / deg

    s_pad = jnp.concatenate([s, jnp.zeros((NW * SPW - N_NODES,), jnp.float32)])
    w1t = jnp.tile(W1[0], 2)
    b1t = jnp.tile(b1, 2)
    flat = _flat_kernel(s_pad, w1t, b1t)
    y = _head_call(Wreg, flat, breg.reshape(Y_DIM, 1))
    return y[:, 0]
